# baseline (speedup 1.0000x reference)
import functools

import jax
import jax.numpy as jnp
from jax import lax
from jax.experimental import pallas as pl
from jax.experimental.pallas import tpu as pltpu
from jax.experimental.pallas import tpu_sc as plsc

_VOCAB1 = 1000001
_VPAD = 1000064
_D = 32
_BATCH = 16384
_HIST = 20
_B = _BATCH * _HIST

_INFO = plsc.get_sparse_core_info()
_NW = _INFO.num_cores * _INFO.num_subcores
_NBLK = _VPAD // 128
_NFULL = _NBLK - 1
_TAIL0 = _NFULL * 128
_NTAIL = _VOCAB1 - _TAIL0
_CHUNK = 128
_NCHUNK = _B // (_NW * _CHUNK)


def _transpose_body(emb_t, tail, out, stage, buf, sem_in, sem_out):
    wid = lax.axis_index("s") * _INFO.num_cores + lax.axis_index("c")

    d16 = lax.iota(jnp.int32, 16)

    def do_block(c):
        pltpu.async_copy(
            emb_t.at[:, pl.ds(c * 128, 128)], stage, sem_in
        ).wait()
        for r in range(128):
            for k in range(2):
                vals = plsc.load_gather(
                    stage, [d16 + (16 * k), jnp.full((16,), r, jnp.int32)]
                )
                buf[pl.ds(r * _D + 16 * k, 16)] = vals
        pltpu.async_copy(
            buf, out.at[pl.ds(c * 128 * _D, 128 * _D)], sem_out
        ).wait()

    def loop_body(t, carry):
        c = wid + t * _NW
        @pl.when(c < _NFULL)
        def _():
            do_block(c)
        return carry

    lax.fori_loop(0, (_NFULL + _NW - 1) // _NW, loop_body, 0)

    @pl.when(wid == 0)
    def _():
        pltpu.sync_copy(tail, buf.at[pl.ds(0, _NTAIL * _D)])
        pltpu.sync_copy(
            buf.at[pl.ds(0, _NTAIL * _D)],
            out.at[pl.ds(_TAIL0 * _D, _NTAIL * _D)],
        )


@jax.jit
def _transpose(emb_t, tail):
    mesh = plsc.VectorSubcoreMesh(core_axis_name="c", subcore_axis_name="s")
    k = pl.kernel(
        _transpose_body,
        out_type=jax.ShapeDtypeStruct((_VPAD * _D,), jnp.float32),
        mesh=mesh,
        scratch_types=[
            pltpu.VMEM((32, 128), jnp.float32),
            pltpu.VMEM((128 * _D,), jnp.float32),
            pltpu.SemaphoreType.DMA,
            pltpu.SemaphoreType.DMA,
        ],
        compiler_params=pltpu.CompilerParams(
            use_tc_tiling_on_sc=True, needs_layout_passes=False
        ),
    )
    return k(emb_t, tail)


def _gather_body(table, idx_hbm, out, idx_v, rows, tbuf, sem_i, sem_g, sem_o):
    wid = lax.axis_index("s") * _INFO.num_cores + lax.axis_index("c")
    base = wid * _NCHUNK * _CHUNK
    pltpu.sync_copy(idx_hbm.at[pl.ds(base, _NCHUNK * _CHUNK)], idx_v)

    i16 = lax.iota(jnp.int32, 16)

    def do_chunk(t):
        p = wid * _NCHUNK + t
        h = p // 128
        ib = p % 128
        pltpu.async_copy(
            table.at[idx_v.at[pl.ds(t * _CHUNK, _CHUNK)]], rows, sem_g
        ).wait()
        for d in range(_D):
            for k in range(8):
                vals = plsc.load_gather(
                    rows,
                    [i16 + (16 * k), jnp.full((16,), d, jnp.int32)],
                )
                tbuf[d // 8, d % 8, pl.ds(16 * k, 16)] = vals
        for db in range(4):
            pltpu.async_copy(tbuf.at[db], out.at[h, db, ib], sem_o).wait()

    def loop_body(t, carry):
        do_chunk(t)
        return carry

    lax.fori_loop(0, _NCHUNK, loop_body, 0)


@jax.jit
def _gather(table_rm, idx_flat):
    mesh = plsc.VectorSubcoreMesh(core_axis_name="c", subcore_axis_name="s")
    k = pl.kernel(
        _gather_body,
        out_type=jax.ShapeDtypeStruct((_HIST, 4, 128, 8, 128), jnp.float32),
        mesh=mesh,
        scratch_types=[
            pltpu.VMEM((_NCHUNK * _CHUNK,), jnp.int32),
            pltpu.VMEM((_CHUNK, _D), jnp.float32),
            pltpu.VMEM((4, 8, 128), jnp.float32),
            pltpu.SemaphoreType.DMA,
            pltpu.SemaphoreType.DMA,
            pltpu.SemaphoreType.DMA,
        ],
        compiler_params=pltpu.CompilerParams(
            use_tc_tiling_on_sc=False, needs_layout_passes=False
        ),
    )
    return k(table_rm, idx_flat)


@jax.jit
def kernel(inputs, embeddings):
    emb_t = embeddings.T
    tail = lax.slice(embeddings, (_TAIL0, 0), (_VOCAB1, _D)).reshape(-1)
    scratch = _transpose(emb_t, tail)
    table_rm = scratch.reshape(_VPAD, _D)
    idx_flat = inputs.T.reshape(-1)
    o = _gather(table_rm, idx_flat)
    return o.transpose(2, 4, 0, 1, 3).reshape(_BATCH, _HIST, _D)


# speedup vs baseline: 1.4062x; 1.4062x over previous
import functools

import jax
import jax.numpy as jnp
from jax import lax
from jax.experimental import pallas as pl
from jax.experimental.pallas import tpu as pltpu
from jax.experimental.pallas import tpu_sc as plsc

_VOCAB1 = 1000001
_VPAD = 1000064
_D = 32
_BATCH = 16384
_HIST = 20
_B = _BATCH * _HIST

_INFO = plsc.get_sparse_core_info()
_NW = _INFO.num_cores * _INFO.num_subcores

_NSUPER = 7812
_TAIL0 = _NSUPER * 128
_NTAIL = _VOCAB1 - _TAIL0

_CHUNK = 128


def _transpose_body(emb_t, tail, out, st0, st1, bf0, bf1, gi0, gi1, go0, go1):
    wid = lax.axis_index("s") * _INFO.num_cores + lax.axis_index("c")
    d16a = lax.iota(jnp.int32, 16)
    d16b = d16a + 16
    stage = (st0, st1)
    buf = (bf0, bf1)
    sin = (gi0, gi1)
    sout = (go0, go1)
    nwords = 128 * _D

    def start_in(s, par):
        pltpu.async_copy(
            emb_t.at[:, pl.ds(s * 128, 128)], stage[par], sin[par]
        )

    def handle(s, par, t):
        @pl.when(s < _NSUPER)
        def _():
            pltpu.make_async_copy(
                emb_t.at[:, pl.ds(0, 128)], stage[par], sin[par]
            ).wait()
            @pl.when(t > 0)
            def _():
                pltpu.make_async_copy(
                    buf[par], out.at[pl.ds(0, nwords)], sout[par]
                ).wait()

            for r in range(128):
                src = jnp.full((16,), r, jnp.int32)
                va = plsc.load_gather(stage[par], [d16a, src])
                vb = plsc.load_gather(stage[par], [d16b, src])
                buf[par][pl.ds(r * _D, 16)] = va
                buf[par][pl.ds(r * _D + 16, 16)] = vb
            pltpu.async_copy(
                buf[par], out.at[pl.ds(s * nwords, nwords)], sout[par]
            )
            @pl.when(s + 2 * _NW < _NSUPER)
            def _():
                start_in(s + 2 * _NW, par)

    @pl.when(wid < _NSUPER)
    def _():
        start_in(wid, 0)

    @pl.when(wid + _NW < _NSUPER)
    def _():
        start_in(wid + _NW, 1)

    def loop_body(t, carry):
        handle(wid + 2 * t * _NW, 0, t)
        handle(wid + (2 * t + 1) * _NW, 1, t)
        return carry

    niter = (_NSUPER + 2 * _NW - 1) // (2 * _NW)
    lax.fori_loop(0, niter, loop_body, 0)

    @pl.when(wid < _NSUPER)
    def _():
        pltpu.make_async_copy(buf[0], out.at[pl.ds(0, nwords)], sout[0]).wait()

    @pl.when(wid + _NW < _NSUPER)
    def _():
        pltpu.make_async_copy(buf[1], out.at[pl.ds(0, nwords)], sout[1]).wait()

    @pl.when(wid == 0)
    def _():
        pltpu.sync_copy(tail, bf0.at[pl.ds(0, _NTAIL * _D)])
        pltpu.sync_copy(
            bf0.at[pl.ds(0, _NTAIL * _D)],
            out.at[pl.ds(_TAIL0 * _D, _NTAIL * _D)],
        )


@jax.jit
def _transpose(emb_t, tail):
    mesh = plsc.VectorSubcoreMesh(core_axis_name="c", subcore_axis_name="s")
    k = pl.kernel(
        _transpose_body,
        out_type=jax.ShapeDtypeStruct((_VPAD * _D,), jnp.float32),
        mesh=mesh,
        scratch_types=[
            pltpu.VMEM((32, 128), jnp.float32),
            pltpu.VMEM((32, 128), jnp.float32),
            pltpu.VMEM((128 * _D,), jnp.float32),
            pltpu.VMEM((128 * _D,), jnp.float32),
            pltpu.SemaphoreType.DMA,
            pltpu.SemaphoreType.DMA,
            pltpu.SemaphoreType.DMA,
            pltpu.SemaphoreType.DMA,
        ],
        compiler_params=pltpu.CompilerParams(
            use_tc_tiling_on_sc=True, needs_layout_passes=False
        ),
    )
    return k(emb_t, tail)


def _gather_body(
    table, idx_hbm, out, idx_v, rw0, rw1, tb0, tb1, gi0, gi1, go0, go1
):
    wid = lax.axis_index("s") * _INFO.num_cores + lax.axis_index("c")
    nper = _B // _NW
    nchunk = nper // _CHUNK
    base = wid * nper
    pltpu.sync_copy(idx_hbm.at[pl.ds(base, nper)], idx_v)
    i16 = lax.iota(jnp.int32, 16)
    rows = (rw0, rw1)
    tbuf = (tb0, tb1)
    sin = (gi0, gi1)
    sout = (go0, go1)

    def start_in(t, par):
        pltpu.async_copy(
            table.at[idx_v.at[pl.ds(t * _CHUNK, _CHUNK)]], rows[par], sin[par]
        )

    def wait_out(par):
        for db in range(4):
            pltpu.make_async_copy(
                tbuf[par].at[db], out.at[0, db, 0], sout[par]
            ).wait()

    def handle(t, par, first):
        pltpu.make_async_copy(
            table.at[idx_v.at[pl.ds(0, _CHUNK)]], rows[par], sin[par]
        ).wait()
        @pl.when(jnp.logical_not(first))
        def _():
            wait_out(par)

        for d in range(_D):
            dv = jnp.full((16,), d, jnp.int32)
            for k in range(8):
                vals = plsc.load_gather(rows[par], [i16 + (16 * k), dv])
                tbuf[par][d // 8, d % 8, pl.ds(16 * k, 16)] = vals
        q = wid * nchunk + t
        h = q // 128
        ib = q % 128
        for db in range(4):
            pltpu.async_copy(
                tbuf[par].at[db], out.at[h, db, ib], sout[par]
            )
        nxt = t + 2
        @pl.when(nxt < nchunk)
        def _():
            start_in(nxt, par)

    start_in(0, 0)
    start_in(1, 1)

    def loop_body(t, carry):
        handle(2 * t, 0, t == 0)
        handle(2 * t + 1, 1, t == 0)
        return carry

    lax.fori_loop(0, nchunk // 2, loop_body, 0)
    wait_out(0)
    wait_out(1)


@jax.jit
def _gather(table_rm, idx_flat):
    mesh = plsc.VectorSubcoreMesh(core_axis_name="c", subcore_axis_name="s")
    k = pl.kernel(
        _gather_body,
        out_type=jax.ShapeDtypeStruct((_HIST, 4, 128, 8, 128), jnp.float32),
        mesh=mesh,
        scratch_types=[
            pltpu.VMEM((_B // _NW,), jnp.int32),
            pltpu.VMEM((_CHUNK, _D), jnp.float32),
            pltpu.VMEM((_CHUNK, _D), jnp.float32),
            pltpu.VMEM((4, 8, 128), jnp.float32),
            pltpu.VMEM((4, 8, 128), jnp.float32),
            pltpu.SemaphoreType.DMA,
            pltpu.SemaphoreType.DMA,
            pltpu.SemaphoreType.DMA,
            pltpu.SemaphoreType.DMA,
        ],
        compiler_params=pltpu.CompilerParams(
            use_tc_tiling_on_sc=False, needs_layout_passes=False
        ),
    )
    return k(table_rm, idx_flat)


@jax.jit
def kernel(inputs, embeddings):
    emb_t = embeddings.T
    tail = lax.slice(embeddings, (_TAIL0, 0), (_VOCAB1, _D)).reshape(-1)
    scratch = _transpose(emb_t, tail)
    table_rm = scratch.reshape(_VPAD, _D)
    idx_flat = inputs.T.reshape(-1)
    o = _gather(table_rm, idx_flat)
    return o.transpose(2, 4, 0, 1, 3).reshape(_BATCH, _HIST, _D)


# speedup vs baseline: 1.8691x; 1.3292x over previous
import functools

import jax
import jax.numpy as jnp
from jax import lax
from jax.experimental import pallas as pl
from jax.experimental.pallas import tpu as pltpu
from jax.experimental.pallas import tpu_sc as plsc

_VOCAB1 = 1000001
_VPAD = 1000064
_D = 32
_BATCH = 16384
_HIST = 20
_B = _BATCH * _HIST

_INFO = plsc.get_sparse_core_info()
_NW = _INFO.num_cores * _INFO.num_subcores

_NSUPER = 7812
_TAIL0 = _NSUPER * 128
_NTAIL = _VOCAB1 - _TAIL0

_CHUNK = 128


def _transpose_body(emb_t, tail, out, st0, st1, bf0, bf1, gi0, gi1, go0, go1):
    wid = lax.axis_index("s") * _INFO.num_cores + lax.axis_index("c")
    d16a = lax.iota(jnp.int32, 16)
    d16b = d16a + 16
    stage = (st0, st1)
    buf = (bf0, bf1)
    sin = (gi0, gi1)
    sout = (go0, go1)
    nwords = 128 * _D

    def start_in(s, par):
        pltpu.async_copy(
            emb_t.at[:, pl.ds(s * 128, 128)], stage[par], sin[par]
        )

    def handle(s, par, t):
        @pl.when(s < _NSUPER)
        def _():
            pltpu.make_async_copy(
                emb_t.at[:, pl.ds(0, 128)], stage[par], sin[par]
            ).wait()
            @pl.when(t > 0)
            def _():
                pltpu.make_async_copy(
                    buf[par], out.at[pl.ds(0, nwords)], sout[par]
                ).wait()

            for g in range(0, 128, 8):
                vals = []
                for r in range(g, g + 8):
                    src = jnp.full((16,), r, jnp.int32)
                    vals.append(
                        (
                            plsc.load_gather(stage[par], [d16a, src]),
                            plsc.load_gather(stage[par], [d16b, src]),
                        )
                    )
                for i, r in enumerate(range(g, g + 8)):
                    buf[par][pl.ds(r * _D, 16)] = vals[i][0]
                    buf[par][pl.ds(r * _D + 16, 16)] = vals[i][1]
            pltpu.async_copy(
                buf[par], out.at[pl.ds(s * nwords, nwords)], sout[par]
            )
            @pl.when(s + 2 * _NW < _NSUPER)
            def _():
                start_in(s + 2 * _NW, par)

    @pl.when(wid < _NSUPER)
    def _():
        start_in(wid, 0)

    @pl.when(wid + _NW < _NSUPER)
    def _():
        start_in(wid + _NW, 1)

    def loop_body(t, carry):
        handle(wid + 2 * t * _NW, 0, t)
        handle(wid + (2 * t + 1) * _NW, 1, t)
        return carry

    niter = (_NSUPER + 2 * _NW - 1) // (2 * _NW)
    lax.fori_loop(0, niter, loop_body, 0)

    @pl.when(wid < _NSUPER)
    def _():
        pltpu.make_async_copy(buf[0], out.at[pl.ds(0, nwords)], sout[0]).wait()

    @pl.when(wid + _NW < _NSUPER)
    def _():
        pltpu.make_async_copy(buf[1], out.at[pl.ds(0, nwords)], sout[1]).wait()

    @pl.when(wid == 0)
    def _():
        pltpu.sync_copy(tail, bf0.at[pl.ds(0, _NTAIL * _D)])
        pltpu.sync_copy(
            bf0.at[pl.ds(0, _NTAIL * _D)],
            out.at[pl.ds(_TAIL0 * _D, _NTAIL * _D)],
        )


@jax.jit
def _transpose(emb_t, tail):
    mesh = plsc.VectorSubcoreMesh(core_axis_name="c", subcore_axis_name="s")
    k = pl.kernel(
        _transpose_body,
        out_type=jax.ShapeDtypeStruct((_VPAD * _D,), jnp.float32),
        mesh=mesh,
        scratch_types=[
            pltpu.VMEM((32, 128), jnp.float32),
            pltpu.VMEM((32, 128), jnp.float32),
            pltpu.VMEM((128 * _D,), jnp.float32),
            pltpu.VMEM((128 * _D,), jnp.float32),
            pltpu.SemaphoreType.DMA,
            pltpu.SemaphoreType.DMA,
            pltpu.SemaphoreType.DMA,
            pltpu.SemaphoreType.DMA,
        ],
        compiler_params=pltpu.CompilerParams(
            use_tc_tiling_on_sc=True, needs_layout_passes=False
        ),
    )
    return k(emb_t, tail)


def _gather_body(
    table, idx_hbm, out, idx_v, rw0, rw1, tb0, tb1, gi0, gi1, go0, go1
):
    wid = lax.axis_index("s") * _INFO.num_cores + lax.axis_index("c")
    nper = _B // _NW
    nchunk = nper // _CHUNK
    base = wid * nper
    pltpu.sync_copy(idx_hbm.at[pl.ds(base, nper)], idx_v)
    i16 = lax.iota(jnp.int32, 16)
    rows = (rw0, rw1)
    tbuf = (tb0, tb1)
    sin = (gi0, gi1)
    sout = (go0, go1)

    def start_in(t, par):
        pltpu.async_copy(
            table.at[idx_v.at[pl.ds(t * _CHUNK, _CHUNK)]], rows[par], sin[par]
        )

    def wait_out(par):
        for db in range(4):
            pltpu.make_async_copy(
                tbuf[par].at[db], out.at[0, db, 0], sout[par]
            ).wait()

    def handle(t, par, first):
        pltpu.make_async_copy(
            table.at[idx_v.at[pl.ds(0, _CHUNK)]], rows[par], sin[par]
        ).wait()
        @pl.when(jnp.logical_not(first))
        def _():
            wait_out(par)

        for d in range(_D):
            dv = jnp.full((16,), d, jnp.int32)
            vals = [
                plsc.load_gather(rows[par], [i16 + (16 * k), dv])
                for k in range(8)
            ]
            for k in range(8):
                tbuf[par][d // 8, d % 8, pl.ds(16 * k, 16)] = vals[k]
        q = wid * nchunk + t
        h = q // 128
        ib = q % 128
        for db in range(4):
            pltpu.async_copy(
                tbuf[par].at[db], out.at[h, db, ib], sout[par]
            )
        nxt = t + 2
        @pl.when(nxt < nchunk)
        def _():
            start_in(nxt, par)

    start_in(0, 0)
    start_in(1, 1)

    def loop_body(t, carry):
        handle(2 * t, 0, t == 0)
        handle(2 * t + 1, 1, t == 0)
        return carry

    lax.fori_loop(0, nchunk // 2, loop_body, 0)
    wait_out(0)
    wait_out(1)


@jax.jit
def _gather(table_rm, idx_flat):
    mesh = plsc.VectorSubcoreMesh(core_axis_name="c", subcore_axis_name="s")
    k = pl.kernel(
        _gather_body,
        out_type=jax.ShapeDtypeStruct((_HIST, 4, 128, 8, 128), jnp.float32),
        mesh=mesh,
        scratch_types=[
            pltpu.VMEM((_B // _NW,), jnp.int32),
            pltpu.VMEM((_CHUNK, _D), jnp.float32),
            pltpu.VMEM((_CHUNK, _D), jnp.float32),
            pltpu.VMEM((4, 8, 128), jnp.float32),
            pltpu.VMEM((4, 8, 128), jnp.float32),
            pltpu.SemaphoreType.DMA,
            pltpu.SemaphoreType.DMA,
            pltpu.SemaphoreType.DMA,
            pltpu.SemaphoreType.DMA,
        ],
        compiler_params=pltpu.CompilerParams(
            use_tc_tiling_on_sc=False, needs_layout_passes=False
        ),
    )
    return k(table_rm, idx_flat)


@jax.jit
def kernel(inputs, embeddings):
    emb_t = embeddings.T
    tail = lax.slice(embeddings, (_TAIL0, 0), (_VOCAB1, _D)).reshape(-1)
    scratch = _transpose(emb_t, tail)
    table_rm = scratch.reshape(_VPAD, _D)
    idx_flat = inputs.T.reshape(-1)
    o = _gather(table_rm, idx_flat)
    return o.transpose(2, 4, 0, 1, 3).reshape(_BATCH, _HIST, _D)


# speedup vs baseline: 2.1617x; 1.1565x over previous
import functools

import jax
import jax.numpy as jnp
from jax import lax
from jax.experimental import pallas as pl
from jax.experimental.pallas import tpu as pltpu
from jax.experimental.pallas import tpu_sc as plsc

_VOCAB1 = 1000001
_VPAD = 1000064
_D = 32
_BATCH = 16384
_HIST = 20
_B = _BATCH * _HIST

_INFO = plsc.get_sparse_core_info()
_NW = _INFO.num_cores * _INFO.num_subcores

_NSUPER = 7812
_TAIL0 = _NSUPER * 128
_NTAIL = _VOCAB1 - _TAIL0

_CHUNK = 128


def _transpose_body(emb_t, tail, out, st0, st1, bf0, bf1, gi0, gi1, go0, go1):
    wid = lax.axis_index("s") * _INFO.num_cores + lax.axis_index("c")
    d16a = lax.iota(jnp.int32, 16)
    d16b = d16a + 16
    stage = (st0, st1)
    buf = (bf0, bf1)
    sin = (gi0, gi1)
    sout = (go0, go1)
    nwords = 128 * _D

    def start_in(s, par):
        pltpu.async_copy(
            emb_t.at[:, pl.ds(s * 128, 128)],
            stage[par].at[:, pl.ds(0, 128)],
            sin[par],
        )

    def handle(s, par, t):
        @pl.when(s < _NSUPER)
        def _():
            pltpu.make_async_copy(
                emb_t.at[:, pl.ds(0, 128)],
                stage[par].at[:, pl.ds(0, 128)],
                sin[par],
            ).wait()
            @pl.when(t > 0)
            def _():
                pltpu.make_async_copy(
                    buf[par], out.at[pl.ds(0, nwords)], sout[par]
                ).wait()

            for g in range(0, 128, 8):
                vals = []
                for r in range(g, g + 8):
                    src = jnp.full((16,), r, jnp.int32)
                    vals.append(
                        (
                            plsc.load_gather(stage[par], [d16a, src]),
                            plsc.load_gather(stage[par], [d16b, src]),
                        )
                    )
                for i, r in enumerate(range(g, g + 8)):
                    buf[par][pl.ds(r * _D, 16)] = vals[i][0]
                    buf[par][pl.ds(r * _D + 16, 16)] = vals[i][1]
            pltpu.async_copy(
                buf[par], out.at[pl.ds(s * nwords, nwords)], sout[par]
            )
            @pl.when(s + 2 * _NW < _NSUPER)
            def _():
                start_in(s + 2 * _NW, par)

    @pl.when(wid < _NSUPER)
    def _():
        start_in(wid, 0)

    @pl.when(wid + _NW < _NSUPER)
    def _():
        start_in(wid + _NW, 1)

    def loop_body(t, carry):
        handle(wid + 2 * t * _NW, 0, t)
        handle(wid + (2 * t + 1) * _NW, 1, t)
        return carry

    niter = (_NSUPER + 2 * _NW - 1) // (2 * _NW)
    lax.fori_loop(0, niter, loop_body, 0)

    @pl.when(wid < _NSUPER)
    def _():
        pltpu.make_async_copy(buf[0], out.at[pl.ds(0, nwords)], sout[0]).wait()

    @pl.when(wid + _NW < _NSUPER)
    def _():
        pltpu.make_async_copy(buf[1], out.at[pl.ds(0, nwords)], sout[1]).wait()

    @pl.when(wid == 0)
    def _():
        pltpu.sync_copy(tail, bf0.at[pl.ds(0, _NTAIL * _D)])
        pltpu.sync_copy(
            bf0.at[pl.ds(0, _NTAIL * _D)],
            out.at[pl.ds(_TAIL0 * _D, _NTAIL * _D)],
        )


@jax.jit
def _transpose(emb_t, tail):
    mesh = plsc.VectorSubcoreMesh(core_axis_name="c", subcore_axis_name="s")
    k = pl.kernel(
        _transpose_body,
        out_type=jax.ShapeDtypeStruct((_VPAD * _D,), jnp.float32),
        mesh=mesh,
        scratch_types=[
            pltpu.VMEM((32, 129), jnp.float32),
            pltpu.VMEM((32, 129), jnp.float32),
            pltpu.VMEM((128 * _D,), jnp.float32),
            pltpu.VMEM((128 * _D,), jnp.float32),
            pltpu.SemaphoreType.DMA,
            pltpu.SemaphoreType.DMA,
            pltpu.SemaphoreType.DMA,
            pltpu.SemaphoreType.DMA,
        ],
        compiler_params=pltpu.CompilerParams(
            use_tc_tiling_on_sc=True, needs_layout_passes=False
        ),
    )
    return k(emb_t, tail)


def _gather_body(
    table, idx_hbm, out, idx_v, rw0, rw1, tb0, tb1, gi0, gi1, go0, go1
):
    wid = lax.axis_index("s") * _INFO.num_cores + lax.axis_index("c")
    nper = _B // _NW
    nchunk = nper // _CHUNK
    base = wid * nper
    pltpu.sync_copy(idx_hbm.at[pl.ds(base, nper)], idx_v)
    i16 = lax.iota(jnp.int32, 16)
    rows = (rw0, rw1)
    tbuf = (tb0, tb1)
    sin = (gi0, gi1)
    sout = (go0, go1)

    def start_in(t, par):
        pltpu.async_copy(
            table.at[idx_v.at[pl.ds(t * _CHUNK, _CHUNK)]], rows[par], sin[par]
        )

    def wait_out(par):
        for db in range(4):
            pltpu.make_async_copy(
                tbuf[par].at[pl.ds(8 * db, 8), pl.ds(0, 128)],
                out.at[0, db, 0],
                sout[par],
            ).wait()

    def handle(t, par, first):
        pltpu.make_async_copy(
            table.at[idx_v.at[pl.ds(0, _CHUNK)]], rows[par], sin[par]
        ).wait()
        @pl.when(jnp.logical_not(first))
        def _():
            wait_out(par)

        for g in range(0, 128, 8):
            vals = []
            for il in range(g, g + 8):
                vals.append(
                    (rows[par][il, pl.ds(0, 16)], rows[par][il, pl.ds(16, 16)])
                )
            for i, il in enumerate(range(g, g + 8)):
                ilv = jnp.full((16,), il, jnp.int32)
                plsc.store_scatter(tbuf[par], [i16, ilv], vals[i][0])
                plsc.store_scatter(tbuf[par], [i16 + 16, ilv], vals[i][1])
        q = wid * nchunk + t
        h = q // 128
        ib = q % 128
        for db in range(4):
            pltpu.async_copy(
                tbuf[par].at[pl.ds(8 * db, 8), pl.ds(0, 128)],
                out.at[h, db, ib],
                sout[par],
            )
        nxt = t + 2
        @pl.when(nxt < nchunk)
        def _():
            start_in(nxt, par)

    start_in(0, 0)
    start_in(1, 1)

    def loop_body(t, carry):
        handle(2 * t, 0, t == 0)
        handle(2 * t + 1, 1, t == 0)
        return carry

    lax.fori_loop(0, nchunk // 2, loop_body, 0)
    wait_out(0)
    wait_out(1)


@jax.jit
def _gather(table_rm, idx_flat):
    mesh = plsc.VectorSubcoreMesh(core_axis_name="c", subcore_axis_name="s")
    k = pl.kernel(
        _gather_body,
        out_type=jax.ShapeDtypeStruct((_HIST, 4, 128, 8, 128), jnp.float32),
        mesh=mesh,
        scratch_types=[
            pltpu.VMEM((_B // _NW,), jnp.int32),
            pltpu.VMEM((_CHUNK, _D), jnp.float32),
            pltpu.VMEM((_CHUNK, _D), jnp.float32),
            pltpu.VMEM((_D, 129), jnp.float32),
            pltpu.VMEM((_D, 129), jnp.float32),
            pltpu.SemaphoreType.DMA,
            pltpu.SemaphoreType.DMA,
            pltpu.SemaphoreType.DMA,
            pltpu.SemaphoreType.DMA,
        ],
        compiler_params=pltpu.CompilerParams(
            use_tc_tiling_on_sc=False, needs_layout_passes=False
        ),
    )
    return k(table_rm, idx_flat)


@jax.jit
def kernel(inputs, embeddings):
    emb_t = embeddings.T
    tail = lax.slice(embeddings, (_TAIL0, 0), (_VOCAB1, _D)).reshape(-1)
    scratch = _transpose(emb_t, tail)
    table_rm = scratch.reshape(_VPAD, _D)
    idx_flat = inputs.T.reshape(-1)
    o = _gather(table_rm, idx_flat)
    return o.transpose(2, 4, 0, 1, 3).reshape(_BATCH, _HIST, _D)


# speedup vs baseline: 2.4067x; 1.1133x over previous
import functools

import jax
import jax.numpy as jnp
from jax import lax
from jax.experimental import pallas as pl
from jax.experimental.pallas import tpu as pltpu
from jax.experimental.pallas import tpu_sc as plsc

_VOCAB1 = 1000001
_VPAD = 1000064
_D = 32
_BATCH = 16384
_HIST = 20
_B = _BATCH * _HIST

_INFO = plsc.get_sparse_core_info()
_NW = _INFO.num_cores * _INFO.num_subcores

_NSUPER = 7812
_TAIL0 = _NSUPER * 128
_NTAIL = _VOCAB1 - _TAIL0

_CHUNK = 128


def _transpose_body(emb_t, tail, out, st0, st1, bf0, bf1, gi0, gi1, go0, go1):
    wid = lax.axis_index("s") * _INFO.num_cores + lax.axis_index("c")
    d16a = lax.iota(jnp.int32, 16)
    d16b = d16a + 16
    stage = (st0, st1)
    buf = (bf0, bf1)
    sin = (gi0, gi1)
    sout = (go0, go1)
    nwords = 128 * _D

    def start_in(s, par):
        pltpu.async_copy(
            emb_t.at[:, pl.ds(s * 128, 128)], stage[par], sin[par]
        )

    def handle(s, par, t):
        @pl.when(s < _NSUPER)
        def _():
            pltpu.make_async_copy(
                emb_t.at[:, pl.ds(0, 128)], stage[par], sin[par]
            ).wait()
            @pl.when(t > 0)
            def _():
                pltpu.make_async_copy(
                    buf[par], out.at[pl.ds(0, nwords)], sout[par]
                ).wait()

            for k in range(8):
                cv = d16a * _D + (16 * k * _D)
                for g in range(0, _D, 8):
                    vals = [
                        stage[par][d, pl.ds(16 * k, 16)]
                        for d in range(g, g + 8)
                    ]
                    for i, d in enumerate(range(g, g + 8)):
                        plsc.store_scatter(buf[par], [cv + d], vals[i])
            pltpu.async_copy(
                buf[par], out.at[pl.ds(s * nwords, nwords)], sout[par]
            )
            @pl.when(s + 2 * _NW < _NSUPER)
            def _():
                start_in(s + 2 * _NW, par)

    @pl.when(wid < _NSUPER)
    def _():
        start_in(wid, 0)

    @pl.when(wid + _NW < _NSUPER)
    def _():
        start_in(wid + _NW, 1)

    def loop_body(t, carry):
        handle(wid + 2 * t * _NW, 0, t)
        handle(wid + (2 * t + 1) * _NW, 1, t)
        return carry

    niter = (_NSUPER + 2 * _NW - 1) // (2 * _NW)
    lax.fori_loop(0, niter, loop_body, 0)

    @pl.when(wid < _NSUPER)
    def _():
        pltpu.make_async_copy(buf[0], out.at[pl.ds(0, nwords)], sout[0]).wait()

    @pl.when(wid + _NW < _NSUPER)
    def _():
        pltpu.make_async_copy(buf[1], out.at[pl.ds(0, nwords)], sout[1]).wait()

    @pl.when(wid == 0)
    def _():
        pltpu.sync_copy(tail, bf0.at[pl.ds(0, _NTAIL * _D)])
        pltpu.sync_copy(
            bf0.at[pl.ds(0, _NTAIL * _D)],
            out.at[pl.ds(_TAIL0 * _D, _NTAIL * _D)],
        )


@jax.jit
def _transpose(emb_t, tail):
    mesh = plsc.VectorSubcoreMesh(core_axis_name="c", subcore_axis_name="s")
    k = pl.kernel(
        _transpose_body,
        out_type=jax.ShapeDtypeStruct((_VPAD * _D,), jnp.float32),
        mesh=mesh,
        scratch_types=[
            pltpu.VMEM((32, 128), jnp.float32),
            pltpu.VMEM((32, 128), jnp.float32),
            pltpu.VMEM((128 * _D,), jnp.float32),
            pltpu.VMEM((128 * _D,), jnp.float32),
            pltpu.SemaphoreType.DMA,
            pltpu.SemaphoreType.DMA,
            pltpu.SemaphoreType.DMA,
            pltpu.SemaphoreType.DMA,
        ],
        compiler_params=pltpu.CompilerParams(
            use_tc_tiling_on_sc=True, needs_layout_passes=False
        ),
    )
    return k(emb_t, tail)


def _gather_body(
    table, idx_hbm, out, idx_v, rw0, rw1, tb0, tb1, gi0, gi1, go0, go1
):
    wid = lax.axis_index("s") * _INFO.num_cores + lax.axis_index("c")
    nper = _B // _NW
    nchunk = nper // _CHUNK
    base = wid * nper
    pltpu.sync_copy(idx_hbm.at[pl.ds(base, nper)], idx_v)
    i16 = lax.iota(jnp.int32, 16)
    rows = (rw0, rw1)
    tbuf = (tb0, tb1)
    sin = (gi0, gi1)
    sout = (go0, go1)

    def start_in(t, par):
        pltpu.async_copy(
            table.at[idx_v.at[pl.ds(t * _CHUNK, _CHUNK)]], rows[par], sin[par]
        )

    def wait_out(par):
        for db in range(4):
            pltpu.make_async_copy(
                tbuf[par].at[pl.ds(8 * db, 8), pl.ds(0, 128)],
                out.at[0, db, 0],
                sout[par],
            ).wait()

    def handle(t, par, first):
        pltpu.make_async_copy(
            table.at[idx_v.at[pl.ds(0, _CHUNK)]], rows[par], sin[par]
        ).wait()
        @pl.when(jnp.logical_not(first))
        def _():
            wait_out(par)

        for g in range(0, 128, 8):
            vals = []
            for il in range(g, g + 8):
                vals.append(
                    (rows[par][il, pl.ds(0, 16)], rows[par][il, pl.ds(16, 16)])
                )
            for i, il in enumerate(range(g, g + 8)):
                ilv = jnp.full((16,), il, jnp.int32)
                plsc.store_scatter(tbuf[par], [i16, ilv], vals[i][0])
                plsc.store_scatter(tbuf[par], [i16 + 16, ilv], vals[i][1])
        q = wid * nchunk + t
        h = q // 128
        ib = q % 128
        for db in range(4):
            pltpu.async_copy(
                tbuf[par].at[pl.ds(8 * db, 8), pl.ds(0, 128)],
                out.at[h, db, ib],
                sout[par],
            )
        nxt = t + 2
        @pl.when(nxt < nchunk)
        def _():
            start_in(nxt, par)

    start_in(0, 0)
    start_in(1, 1)

    def loop_body(t, carry):
        handle(2 * t, 0, t == 0)
        handle(2 * t + 1, 1, t == 0)
        return carry

    lax.fori_loop(0, nchunk // 2, loop_body, 0)
    wait_out(0)
    wait_out(1)


@jax.jit
def _gather(table_rm, idx_flat):
    mesh = plsc.VectorSubcoreMesh(core_axis_name="c", subcore_axis_name="s")
    k = pl.kernel(
        _gather_body,
        out_type=jax.ShapeDtypeStruct((_HIST, 4, 128, 8, 128), jnp.float32),
        mesh=mesh,
        scratch_types=[
            pltpu.VMEM((_B // _NW,), jnp.int32),
            pltpu.VMEM((_CHUNK, _D), jnp.float32),
            pltpu.VMEM((_CHUNK, _D), jnp.float32),
            pltpu.VMEM((_D, 129), jnp.float32),
            pltpu.VMEM((_D, 129), jnp.float32),
            pltpu.SemaphoreType.DMA,
            pltpu.SemaphoreType.DMA,
            pltpu.SemaphoreType.DMA,
            pltpu.SemaphoreType.DMA,
        ],
        compiler_params=pltpu.CompilerParams(
            use_tc_tiling_on_sc=False, needs_layout_passes=False
        ),
    )
    return k(table_rm, idx_flat)


@jax.jit
def kernel(inputs, embeddings):
    emb_t = embeddings.T
    tail = lax.slice(embeddings, (_TAIL0, 0), (_VOCAB1, _D)).reshape(-1)
    scratch = _transpose(emb_t, tail)
    table_rm = scratch.reshape(_VPAD, _D)
    idx_flat = inputs.T.reshape(-1)
    o = _gather(table_rm, idx_flat)
    return o.transpose(2, 4, 0, 1, 3).reshape(_BATCH, _HIST, _D)
